# baseline (speedup 1.0000x reference)
import functools
import math

import jax
import jax.numpy as jnp
from jax.experimental import pallas as pl
from jax.experimental.pallas import tpu as pltpu

B, T, D, H, E, FF = 1, 2048, 1024, 32, 4, 1536
DH = D // H
EPAD = 128


def _bf(a):
    return a.astype(jnp.bfloat16)


def _proj_kernel(x_ref, pe_ref, wq_ref, wk_ref, wv_ref, wp_ref, uf_ref, vf_ref,
                 qu_ref, qv_ref, k_ref, val_ref, p_ref):
    x = _bf(x_ref[...])
    q = jnp.dot(x, _bf(wq_ref[...]), preferred_element_type=jnp.float32)
    scale = 1.0 / math.sqrt(DH)
    qu_ref[...] = _bf((q + uf_ref[...]) * scale)
    qv_ref[...] = _bf((q + vf_ref[...]) * scale)
    k_ref[...] = _bf(jnp.dot(x, _bf(wk_ref[...]), preferred_element_type=jnp.float32))
    val_ref[...] = _bf(jnp.dot(x, _bf(wv_ref[...]), preferred_element_type=jnp.float32))
    p_ref[...] = _bf(jnp.dot(_bf(pe_ref[...]), _bf(wp_ref[...]),
                             preferred_element_type=jnp.float32))


def _bd_kernel(qv_ref, p_ref, out_ref):
    p = p_ref[0]
    p2 = jnp.concatenate([p[T - 1:T], p[:T - 1]], axis=0)
    m = _bf(jax.lax.dot_general(
        qv_ref[0], p2, (((1,), (1,)), ((), ())),
        preferred_element_type=jnp.float32))
    out_ref[0, :, 0:T] = m
    out_ref[0, :, T:T + 1] = m[:, 0:1]


def _attn_kernel(qu_ref, k_ref, v_ref, bd_hbm, out_ref, bd_vmem, sem):
    h = pl.program_id(0)
    i = pl.program_id(1)
    BQ = out_ref.shape[1]
    cp = pltpu.make_async_copy(
        bd_hbm.at[h, pl.ds(i * BQ, BQ + 8), :], bd_vmem, sem)
    cp.start()
    ac = jax.lax.dot_general(
        qu_ref[0], k_ref[0], (((1,), (1,)), ((), ())),
        preferred_element_type=jnp.float32)
    cp.wait()
    row_t = i * BQ + jax.lax.broadcasted_iota(jnp.int32, ac.shape, 0)
    col_s = jax.lax.broadcasted_iota(jnp.int32, ac.shape, 1)
    bdv = bd_vmem[pl.ds(1, BQ), :]
    bd = jnp.where(col_s == row_t + 1, 0.0, bdv.astype(jnp.float32))
    s = ac + bd
    m = jnp.max(s, axis=-1, keepdims=True)
    e = jnp.exp(s - m)
    a = _bf(e / jnp.sum(e, axis=-1, keepdims=True))
    out_ref[0] = _bf(jnp.dot(a, v_ref[0], preferred_element_type=jnp.float32))


def _postattn_kernel(ao_ref, wo_ref, x_ref, g1_ref, b1_ref, x1_ref):
    o = jnp.dot(ao_ref[...], _bf(wo_ref[...]), preferred_element_type=jnp.float32)
    o = o + x_ref[...]
    mu = jnp.mean(o, axis=-1, keepdims=True)
    var = jnp.mean((o - mu) ** 2, axis=-1, keepdims=True)
    x1_ref[...] = (o - mu) * jax.lax.rsqrt(var + 1e-5) * g1_ref[...] + b1_ref[...]


def _router_kernel(x1_ref, wg_ref, gates_ref, loss_ref, dsum, tsum):
    i = pl.program_id(0)
    n = pl.num_programs(0)
    logits = jnp.dot(x1_ref[...], wg_ref[...], preferred_element_type=jnp.float32)
    lane = jax.lax.broadcasted_iota(jnp.int32, logits.shape, 1)
    emask = lane < E
    lm = jnp.where(emask, logits, -1e30)
    mx = jnp.max(lm, axis=-1, keepdims=True)
    p = jnp.exp(lm - mx)
    p = jnp.where(emask, p, 0.0)
    p = p / jnp.sum(p, axis=-1, keepdims=True)
    m1 = jnp.max(p, axis=-1, keepdims=True)
    is1 = p >= m1
    l1 = jnp.min(jnp.where(is1, lane, EPAD), axis=-1, keepdims=True)
    oh1 = lane == l1
    p2 = jnp.where(oh1, -1.0, p)
    m2 = jnp.max(p2, axis=-1, keepdims=True)
    is2 = p2 >= m2
    l2 = jnp.min(jnp.where(is2, lane, EPAD), axis=-1, keepdims=True)
    oh2 = lane == l2
    gates_ref[...] = jnp.where(oh1 | oh2, p, 0.0)

    pd = jnp.sum(p, axis=0, keepdims=True)
    td = jnp.sum(oh1.astype(jnp.float32), axis=0, keepdims=True)

    @pl.when(i == 0)
    def _():
        dsum[...] = pd
        tsum[...] = td
        loss_ref[...] = jnp.zeros_like(loss_ref)

    @pl.when(i > 0)
    def _():
        dsum[...] = dsum[...] + pd
        tsum[...] = tsum[...] + td

    @pl.when(i == n - 1)
    def _():
        density = dsum[...] / jnp.float32(B * T)
        top1 = tsum[...] / jnp.float32(B * T)
        loss_ref[...] = (jnp.float32(0.01) * E *
                         jnp.sum(density * top1)).reshape(1, 1)


def _gelu(x):
    c = math.sqrt(2.0 / math.pi)
    return 0.5 * x * (1.0 + jnp.tanh(c * (x + 0.044715 * (x * x * x))))


def _moe_kernel(x1_ref, gates_ref, we1_ref, be1_ref, we2_ref, be2_ref,
                g2_ref, b2_ref, y_ref, acc):
    e = pl.program_id(1)

    @pl.when(e == 0)
    def _():
        acc[...] = jnp.zeros_like(acc)

    x1 = x1_ref[...]
    h = jnp.dot(_bf(x1), we1_ref[0],
                preferred_element_type=jnp.float32) + be1_ref[0]
    h = _gelu(h)
    eo = jnp.dot(_bf(h), we2_ref[0],
                 preferred_element_type=jnp.float32) + be2_ref[0]
    lane = jax.lax.broadcasted_iota(jnp.int32, gates_ref.shape, 1)
    gcol = jnp.sum(jnp.where(lane == e, gates_ref[...], 0.0), axis=-1, keepdims=True)
    acc[...] = acc[...] + gcol * eo

    @pl.when(e == E - 1)
    def _():
        y = acc[...] + x1
        mu = jnp.mean(y, axis=-1, keepdims=True)
        var = jnp.mean((y - mu) ** 2, axis=-1, keepdims=True)
        y_ref[...] = (y - mu) * jax.lax.rsqrt(var + 1e-5) * g2_ref[...] + b2_ref[...]


def _rel_shift(x):
    b, h, t1, t2 = x.shape
    zp = jnp.zeros((b, h, t1, 1), dtype=x.dtype)
    x = jnp.concatenate([zp, x], axis=-1)
    x = x.reshape(b, h, t2 + 1, t1)
    return x[:, :, 1:, :].reshape(b, h, t1, t2)


@functools.partial(jax.jit, static_argnums=())
def kernel(x, pos_enc, Wq, Wk, Wv, Wo, Wpos, u, v, g1, b1, g2, b2, Wg, We1, be1, We2, be2):
    f32 = jnp.float32
    x2 = x[0]
    pe = pos_enc[0]
    uf = u.reshape(1, D)
    vf = v.reshape(1, D)

    BT = 512
    qu, qv, k, val, p = pl.pallas_call(
        _proj_kernel,
        grid=(T // BT,),
        in_specs=[
            pl.BlockSpec((BT, D), lambda i: (i, 0)),
            pl.BlockSpec((BT, D), lambda i: (i, 0)),
            pl.BlockSpec((D, D), lambda i: (0, 0)),
            pl.BlockSpec((D, D), lambda i: (0, 0)),
            pl.BlockSpec((D, D), lambda i: (0, 0)),
            pl.BlockSpec((D, D), lambda i: (0, 0)),
            pl.BlockSpec((1, D), lambda i: (0, 0)),
            pl.BlockSpec((1, D), lambda i: (0, 0)),
        ],
        out_specs=[pl.BlockSpec((BT, D), lambda i: (i, 0))] * 5,
        out_shape=[jax.ShapeDtypeStruct((T, D), jnp.bfloat16)] * 5,
    )(x2, pe, Wq, Wk, Wv, Wpos, uf, vf)

    def heads(a):
        return a.reshape(T, H, DH).transpose(1, 0, 2)

    quh, qvh, kh, vh, ph = heads(qu), heads(qv), heads(k), heads(val), heads(p)

    bd_pad = pl.pallas_call(
        _bd_kernel,
        grid=(H,),
        in_specs=[
            pl.BlockSpec((1, T, DH), lambda h: (h, 0, 0)),
            pl.BlockSpec((1, T, DH), lambda h: (h, 0, 0)),
        ],
        out_specs=pl.BlockSpec((1, T, T + 1), lambda h: (h, 0, 0)),
        out_shape=jax.ShapeDtypeStruct((H, 2 * T, T + 1), jnp.bfloat16),
    )(qvh, ph)

    bd_view = bd_pad.reshape(H, 2 * T * (T + 1) // T, T)

    BQ = 256
    ao = pl.pallas_call(
        _attn_kernel,
        grid=(H, T // BQ),
        in_specs=[
            pl.BlockSpec((1, BQ, DH), lambda h, i: (h, i, 0)),
            pl.BlockSpec((1, T, DH), lambda h, i: (h, 0, 0)),
            pl.BlockSpec((1, T, DH), lambda h, i: (h, 0, 0)),
            pl.BlockSpec(memory_space=pl.ANY),
        ],
        out_specs=pl.BlockSpec((1, BQ, DH), lambda h, i: (h, i, 0)),
        out_shape=jax.ShapeDtypeStruct((H, T, DH), jnp.bfloat16),
        scratch_shapes=[
            pltpu.VMEM((BQ + 8, T), jnp.bfloat16),
            pltpu.SemaphoreType.DMA,
        ],
    )(quh, kh, vh, bd_view)

    ao_td = ao.transpose(1, 0, 2).reshape(T, D)

    x1 = pl.pallas_call(
        _postattn_kernel,
        grid=(T // BT,),
        in_specs=[
            pl.BlockSpec((BT, D), lambda i: (i, 0)),
            pl.BlockSpec((D, D), lambda i: (0, 0)),
            pl.BlockSpec((BT, D), lambda i: (i, 0)),
            pl.BlockSpec((1, D), lambda i: (0, 0)),
            pl.BlockSpec((1, D), lambda i: (0, 0)),
        ],
        out_specs=pl.BlockSpec((BT, D), lambda i: (i, 0)),
        out_shape=jax.ShapeDtypeStruct((T, D), f32),
    )(ao_td, Wo, x2, g1.reshape(1, D), b1.reshape(1, D))

    Wg_pad = jnp.zeros((D, EPAD), f32).at[:, :E].set(Wg)

    gates, loss = pl.pallas_call(
        _router_kernel,
        grid=(T // BT,),
        in_specs=[
            pl.BlockSpec((BT, D), lambda i: (i, 0)),
            pl.BlockSpec((D, EPAD), lambda i: (0, 0)),
        ],
        out_specs=[
            pl.BlockSpec((BT, EPAD), lambda i: (i, 0)),
            pl.BlockSpec((1, 1), lambda i: (0, 0)),
        ],
        out_shape=[
            jax.ShapeDtypeStruct((T, EPAD), f32),
            jax.ShapeDtypeStruct((1, 1), f32),
        ],
        scratch_shapes=[
            pltpu.VMEM((1, EPAD), f32),
            pltpu.VMEM((1, EPAD), f32),
        ],
    )(x1, Wg_pad)

    y = pl.pallas_call(
        _moe_kernel,
        grid=(T // BT, E),
        in_specs=[
            pl.BlockSpec((BT, D), lambda i, e: (i, 0)),
            pl.BlockSpec((BT, EPAD), lambda i, e: (i, 0)),
            pl.BlockSpec((1, D, FF), lambda i, e: (e, 0, 0)),
            pl.BlockSpec((1, 1, FF), lambda i, e: (e, 0, 0)),
            pl.BlockSpec((1, FF, D), lambda i, e: (e, 0, 0)),
            pl.BlockSpec((1, 1, D), lambda i, e: (e, 0, 0)),
            pl.BlockSpec((1, D), lambda i, e: (0, 0)),
            pl.BlockSpec((1, D), lambda i, e: (0, 0)),
        ],
        out_specs=pl.BlockSpec((BT, D), lambda i, e: (i, 0)),
        out_shape=jax.ShapeDtypeStruct((T, D), f32),
        scratch_shapes=[pltpu.VMEM((BT, D), f32)],
    )(x1, gates, We1.astype(jnp.bfloat16), be1.reshape(E, 1, FF),
      We2.astype(jnp.bfloat16), be2.reshape(E, 1, D),
      g2.reshape(1, D), b2.reshape(1, D))

    return (y.reshape(B, T, D), loss[0, 0])


# speedup vs baseline: 3.0362x; 3.0362x over previous
import functools
import math

import jax
import jax.numpy as jnp
from jax.experimental import pallas as pl
from jax.experimental.pallas import tpu as pltpu

B, T, D, H, E, FF = 1, 2048, 1024, 32, 4, 1536
DH = D // H
EPAD = 128


def _bf(a):
    return a.astype(jnp.bfloat16)


def _proj_kernel(x_ref, pe_ref, wq_ref, wk_ref, wv_ref, wp_ref, uf_ref, vf_ref,
                 qu_ref, qv_ref, k_ref, val_ref, p_ref):
    x = _bf(x_ref[...])
    q = jnp.dot(x, _bf(wq_ref[...]), preferred_element_type=jnp.float32)
    scale = 1.0 / math.sqrt(DH)
    qu_ref[...] = _bf((q + uf_ref[...]) * scale)
    qv_ref[...] = _bf((q + vf_ref[...]) * scale)
    k_ref[...] = _bf(jnp.dot(x, _bf(wk_ref[...]), preferred_element_type=jnp.float32))
    val_ref[...] = _bf(jnp.dot(x, _bf(wv_ref[...]), preferred_element_type=jnp.float32))
    p_ref[...] = _bf(jnp.dot(_bf(pe_ref[...]), _bf(wp_ref[...]),
                             preferred_element_type=jnp.float32))


def _bd_kernel(qv_ref, p_ref, out_ref):
    out_ref[0] = _bf(jax.lax.dot_general(
        qv_ref[0], p_ref[0], (((1,), (1,)), ((), ())),
        preferred_element_type=jnp.float32))


def _attn_kernel(qu_ref, k_ref, v_ref, bd_ref, out_ref):
    ac = jax.lax.dot_general(
        qu_ref[0], k_ref[0], (((1,), (1,)), ((), ())),
        preferred_element_type=jnp.float32)
    s = ac + bd_ref[0, 0].astype(jnp.float32)
    m = jnp.max(s, axis=-1, keepdims=True)
    e = jnp.exp(s - m)
    a = _bf(e / jnp.sum(e, axis=-1, keepdims=True))
    out_ref[0] = _bf(jnp.dot(a, v_ref[0], preferred_element_type=jnp.float32))


def _postattn_kernel(ao_ref, wo_ref, x_ref, g1_ref, b1_ref, x1_ref):
    o = jnp.dot(ao_ref[...], _bf(wo_ref[...]), preferred_element_type=jnp.float32)
    o = o + x_ref[...]
    mu = jnp.mean(o, axis=-1, keepdims=True)
    var = jnp.mean((o - mu) ** 2, axis=-1, keepdims=True)
    x1_ref[...] = (o - mu) * jax.lax.rsqrt(var + 1e-5) * g1_ref[...] + b1_ref[...]


def _router_kernel(x1_ref, wg_ref, gates_ref, loss_ref, dsum, tsum):
    i = pl.program_id(0)
    n = pl.num_programs(0)
    logits = jnp.dot(x1_ref[...], wg_ref[...], preferred_element_type=jnp.float32)
    lane = jax.lax.broadcasted_iota(jnp.int32, logits.shape, 1)
    emask = lane < E
    lm = jnp.where(emask, logits, -1e30)
    mx = jnp.max(lm, axis=-1, keepdims=True)
    p = jnp.exp(lm - mx)
    p = jnp.where(emask, p, 0.0)
    p = p / jnp.sum(p, axis=-1, keepdims=True)
    m1 = jnp.max(p, axis=-1, keepdims=True)
    is1 = p >= m1
    l1 = jnp.min(jnp.where(is1, lane, EPAD), axis=-1, keepdims=True)
    oh1 = lane == l1
    p2 = jnp.where(oh1, -1.0, p)
    m2 = jnp.max(p2, axis=-1, keepdims=True)
    is2 = p2 >= m2
    l2 = jnp.min(jnp.where(is2, lane, EPAD), axis=-1, keepdims=True)
    oh2 = lane == l2
    gates_ref[...] = jnp.where(oh1 | oh2, p, 0.0)

    pd = jnp.sum(p, axis=0, keepdims=True)
    td = jnp.sum(oh1.astype(jnp.float32), axis=0, keepdims=True)

    @pl.when(i == 0)
    def _():
        dsum[...] = pd
        tsum[...] = td
        loss_ref[...] = jnp.zeros_like(loss_ref)

    @pl.when(i > 0)
    def _():
        dsum[...] = dsum[...] + pd
        tsum[...] = tsum[...] + td

    @pl.when(i == n - 1)
    def _():
        density = dsum[...] / jnp.float32(B * T)
        top1 = tsum[...] / jnp.float32(B * T)
        loss_ref[...] = (jnp.float32(0.01) * E *
                         jnp.sum(density * top1)).reshape(1, 1)


def _gelu(x):
    c = math.sqrt(2.0 / math.pi)
    return 0.5 * x * (1.0 + jnp.tanh(c * (x + 0.044715 * (x * x * x))))


def _moe_kernel(x1_ref, gates_ref, we1_ref, be1_ref, we2_ref, be2_ref,
                g2_ref, b2_ref, y_ref, acc):
    e = pl.program_id(1)

    @pl.when(e == 0)
    def _():
        acc[...] = jnp.zeros_like(acc)

    x1 = x1_ref[...]
    h = jnp.dot(_bf(x1), we1_ref[0],
                preferred_element_type=jnp.float32) + be1_ref[0]
    h = _gelu(h)
    eo = jnp.dot(_bf(h), we2_ref[0],
                 preferred_element_type=jnp.float32) + be2_ref[0]
    lane = jax.lax.broadcasted_iota(jnp.int32, gates_ref.shape, 1)
    gcol = jnp.sum(jnp.where(lane == e, gates_ref[...], 0.0), axis=-1, keepdims=True)
    acc[...] = acc[...] + gcol * eo

    @pl.when(e == E - 1)
    def _():
        y = acc[...] + x1
        mu = jnp.mean(y, axis=-1, keepdims=True)
        var = jnp.mean((y - mu) ** 2, axis=-1, keepdims=True)
        y_ref[...] = (y - mu) * jax.lax.rsqrt(var + 1e-5) * g2_ref[...] + b2_ref[...]


def _rel_shift(x):
    b, h, t1, t2 = x.shape
    zp = jnp.zeros((b, h, t1, 1), dtype=x.dtype)
    x = jnp.concatenate([zp, x], axis=-1)
    x = x.reshape(b, h, t2 + 1, t1)
    return x[:, :, 1:, :].reshape(b, h, t1, t2)


@functools.partial(jax.jit, static_argnums=())
def kernel(x, pos_enc, Wq, Wk, Wv, Wo, Wpos, u, v, g1, b1, g2, b2, Wg, We1, be1, We2, be2):
    f32 = jnp.float32
    x2 = x[0]
    pe = pos_enc[0]
    uf = u.reshape(1, D)
    vf = v.reshape(1, D)

    BT = 512
    qu, qv, k, val, p = pl.pallas_call(
        _proj_kernel,
        grid=(T // BT,),
        in_specs=[
            pl.BlockSpec((BT, D), lambda i: (i, 0)),
            pl.BlockSpec((BT, D), lambda i: (i, 0)),
            pl.BlockSpec((D, D), lambda i: (0, 0)),
            pl.BlockSpec((D, D), lambda i: (0, 0)),
            pl.BlockSpec((D, D), lambda i: (0, 0)),
            pl.BlockSpec((D, D), lambda i: (0, 0)),
            pl.BlockSpec((1, D), lambda i: (0, 0)),
            pl.BlockSpec((1, D), lambda i: (0, 0)),
        ],
        out_specs=[pl.BlockSpec((BT, D), lambda i: (i, 0))] * 5,
        out_shape=[jax.ShapeDtypeStruct((T, D), jnp.bfloat16)] * 5,
    )(x2, pe, Wq, Wk, Wv, Wpos, uf, vf)

    def heads(a):
        return a.reshape(T, H, DH).transpose(1, 0, 2)

    quh, qvh, kh, vh, ph = heads(qu), heads(qv), heads(k), heads(val), heads(p)

    bd_raw = pl.pallas_call(
        _bd_kernel,
        grid=(H,),
        in_specs=[
            pl.BlockSpec((1, T, DH), lambda h: (h, 0, 0)),
            pl.BlockSpec((1, T, DH), lambda h: (h, 0, 0)),
        ],
        out_specs=pl.BlockSpec((1, T, T), lambda h: (h, 0, 0)),
        out_shape=jax.ShapeDtypeStruct((H, T, T), jnp.bfloat16),
    )(qvh, ph)

    bd = _rel_shift(bd_raw[None])

    BQ = 256
    ao = pl.pallas_call(
        _attn_kernel,
        grid=(H, T // BQ),
        in_specs=[
            pl.BlockSpec((1, BQ, DH), lambda h, i: (h, i, 0)),
            pl.BlockSpec((1, T, DH), lambda h, i: (h, 0, 0)),
            pl.BlockSpec((1, T, DH), lambda h, i: (h, 0, 0)),
            pl.BlockSpec((1, 1, BQ, T), lambda h, i: (0, h, i, 0)),
        ],
        out_specs=pl.BlockSpec((1, BQ, DH), lambda h, i: (h, i, 0)),
        out_shape=jax.ShapeDtypeStruct((H, T, DH), jnp.bfloat16),
    )(quh, kh, vh, bd)

    ao_td = ao.transpose(1, 0, 2).reshape(T, D)

    x1 = pl.pallas_call(
        _postattn_kernel,
        grid=(T // BT,),
        in_specs=[
            pl.BlockSpec((BT, D), lambda i: (i, 0)),
            pl.BlockSpec((D, D), lambda i: (0, 0)),
            pl.BlockSpec((BT, D), lambda i: (i, 0)),
            pl.BlockSpec((1, D), lambda i: (0, 0)),
            pl.BlockSpec((1, D), lambda i: (0, 0)),
        ],
        out_specs=pl.BlockSpec((BT, D), lambda i: (i, 0)),
        out_shape=jax.ShapeDtypeStruct((T, D), f32),
    )(ao_td, Wo, x2, g1.reshape(1, D), b1.reshape(1, D))

    Wg_pad = jnp.zeros((D, EPAD), f32).at[:, :E].set(Wg)

    gates, loss = pl.pallas_call(
        _router_kernel,
        grid=(T // BT,),
        in_specs=[
            pl.BlockSpec((BT, D), lambda i: (i, 0)),
            pl.BlockSpec((D, EPAD), lambda i: (0, 0)),
        ],
        out_specs=[
            pl.BlockSpec((BT, EPAD), lambda i: (i, 0)),
            pl.BlockSpec((1, 1), lambda i: (0, 0)),
        ],
        out_shape=[
            jax.ShapeDtypeStruct((T, EPAD), f32),
            jax.ShapeDtypeStruct((1, 1), f32),
        ],
        scratch_shapes=[
            pltpu.VMEM((1, EPAD), f32),
            pltpu.VMEM((1, EPAD), f32),
        ],
    )(x1, Wg_pad)

    y = pl.pallas_call(
        _moe_kernel,
        grid=(T // BT, E),
        in_specs=[
            pl.BlockSpec((BT, D), lambda i, e: (i, 0)),
            pl.BlockSpec((BT, EPAD), lambda i, e: (i, 0)),
            pl.BlockSpec((1, D, FF), lambda i, e: (e, 0, 0)),
            pl.BlockSpec((1, 1, FF), lambda i, e: (e, 0, 0)),
            pl.BlockSpec((1, FF, D), lambda i, e: (e, 0, 0)),
            pl.BlockSpec((1, 1, D), lambda i, e: (e, 0, 0)),
            pl.BlockSpec((1, D), lambda i, e: (0, 0)),
            pl.BlockSpec((1, D), lambda i, e: (0, 0)),
        ],
        out_specs=pl.BlockSpec((BT, D), lambda i, e: (i, 0)),
        out_shape=jax.ShapeDtypeStruct((T, D), f32),
        scratch_shapes=[pltpu.VMEM((BT, D), f32)],
    )(x1, gates, We1.astype(jnp.bfloat16), be1.reshape(E, 1, FF),
      We2.astype(jnp.bfloat16), be2.reshape(E, 1, D),
      g2.reshape(1, D), b2.reshape(1, D))

    return (y.reshape(B, T, D), loss[0, 0])


# speedup vs baseline: 4.6589x; 1.5344x over previous
import functools
import math

import jax
import jax.numpy as jnp
from jax.experimental import pallas as pl
from jax.experimental.pallas import tpu as pltpu

B, T, D, H, E, FF = 1, 2048, 1024, 32, 4, 1536
DH = D // H
EPAD = 128


def _bf(a):
    return a.astype(jnp.bfloat16)


def _proj_kernel(x_ref, pe_ref, wq_ref, wk_ref, wv_ref, wp_ref, uf_ref, vf_ref,
                 qu_ref, qv_ref, k_ref, val_ref, p_ref):
    x = _bf(x_ref[...])
    q = jnp.dot(x, _bf(wq_ref[...]), preferred_element_type=jnp.float32)
    scale = 1.0 / math.sqrt(DH)
    qu_ref[...] = _bf((q + uf_ref[...]) * scale)
    qv_ref[...] = _bf((q + vf_ref[...]) * scale)
    k_ref[...] = _bf(jnp.dot(x, _bf(wk_ref[...]), preferred_element_type=jnp.float32))
    val_ref[...] = _bf(jnp.dot(x, _bf(wv_ref[...]), preferred_element_type=jnp.float32))
    p_ref[...] = _bf(jnp.dot(_bf(pe_ref[...]), _bf(wp_ref[...]),
                             preferred_element_type=jnp.float32))


def _roll_r(x, r):
    n = x.shape[-1]
    return jnp.concatenate([x[:, n - r:], x[:, :n - r]], axis=1)


def _attn_kernel(qu_ref, qv_ref, k_ref, v_ref, pd_ref, out_ref):
    i = pl.program_id(1)
    BQ = out_ref.shape[1]
    t0 = i * BQ
    ac = jax.lax.dot_general(
        qu_ref[0], k_ref[0], (((1,), (1,)), ((), ())),
        preferred_element_type=jnp.float32)
    ptile = pd_ref[0, pl.ds(T - t0, T), :]
    qv_sl = qv_ref[0, pl.ds(t0, BQ), :]
    x = _bf(jax.lax.dot_general(
        qv_sl, ptile, (((1,), (1,)), ((), ())),
        preferred_element_type=jnp.float32))
    rowbit_src = jax.lax.broadcasted_iota(jnp.int32, (BQ, 1), 0)
    kb = 0
    while (1 << kb) < BQ:
        r = 1 << kb
        rowbit = (rowbit_src >> kb) & 1
        x = jnp.where(rowbit == 1, _roll_r(x, r), x)
        kb += 1
    rmat = x
    qv_x = qv_ref[0, pl.ds(jnp.minimum(t0 + BQ, T - 8), 8), :][0:1]
    mx = _bf(jax.lax.dot_general(
        qv_x, ptile, (((1,), (1,)), ((), ())),
        preferred_element_type=jnp.float32))
    rup = jnp.concatenate([rmat[1:], _roll_r(mx, BQ)], axis=0)
    row_t = t0 + jax.lax.broadcasted_iota(jnp.int32, ac.shape, 0)
    col_s = jax.lax.broadcasted_iota(jnp.int32, ac.shape, 1)
    bd = jnp.where(col_s <= row_t, rmat,
                   jnp.where(col_s == row_t + 1, jnp.bfloat16(0), rup))
    s = ac + bd.astype(jnp.float32)
    m = jnp.max(s, axis=-1, keepdims=True)
    e = jnp.exp(s - m)
    a = _bf(e / jnp.sum(e, axis=-1, keepdims=True))
    out_ref[0] = _bf(jnp.dot(a, v_ref[0], preferred_element_type=jnp.float32))


def _postattn_kernel(ao_ref, wo_ref, x_ref, g1_ref, b1_ref, x1_ref):
    o = jnp.dot(ao_ref[...], _bf(wo_ref[...]), preferred_element_type=jnp.float32)
    o = o + x_ref[...]
    mu = jnp.mean(o, axis=-1, keepdims=True)
    var = jnp.mean((o - mu) ** 2, axis=-1, keepdims=True)
    x1_ref[...] = (o - mu) * jax.lax.rsqrt(var + 1e-5) * g1_ref[...] + b1_ref[...]


def _router_kernel(x1_ref, wg_ref, gates_ref, loss_ref, dsum, tsum):
    i = pl.program_id(0)
    n = pl.num_programs(0)
    logits = jnp.dot(x1_ref[...], wg_ref[...], preferred_element_type=jnp.float32)
    lane = jax.lax.broadcasted_iota(jnp.int32, logits.shape, 1)
    emask = lane < E
    lm = jnp.where(emask, logits, -1e30)
    mx = jnp.max(lm, axis=-1, keepdims=True)
    p = jnp.exp(lm - mx)
    p = jnp.where(emask, p, 0.0)
    p = p / jnp.sum(p, axis=-1, keepdims=True)
    m1 = jnp.max(p, axis=-1, keepdims=True)
    is1 = p >= m1
    l1 = jnp.min(jnp.where(is1, lane, EPAD), axis=-1, keepdims=True)
    oh1 = lane == l1
    p2 = jnp.where(oh1, -1.0, p)
    m2 = jnp.max(p2, axis=-1, keepdims=True)
    is2 = p2 >= m2
    l2 = jnp.min(jnp.where(is2, lane, EPAD), axis=-1, keepdims=True)
    oh2 = lane == l2
    gates_ref[...] = jnp.where(oh1 | oh2, p, 0.0)

    pd = jnp.sum(p, axis=0, keepdims=True)
    td = jnp.sum(oh1.astype(jnp.float32), axis=0, keepdims=True)

    @pl.when(i == 0)
    def _():
        dsum[...] = pd
        tsum[...] = td
        loss_ref[...] = jnp.zeros_like(loss_ref)

    @pl.when(i > 0)
    def _():
        dsum[...] = dsum[...] + pd
        tsum[...] = tsum[...] + td

    @pl.when(i == n - 1)
    def _():
        density = dsum[...] / jnp.float32(B * T)
        top1 = tsum[...] / jnp.float32(B * T)
        loss_ref[...] = (jnp.float32(0.01) * E *
                         jnp.sum(density * top1)).reshape(1, 1)


def _gelu(x):
    c = math.sqrt(2.0 / math.pi)
    return 0.5 * x * (1.0 + jnp.tanh(c * (x + 0.044715 * (x * x * x))))


def _moe_kernel(x1_ref, gates_ref, we1_ref, be1_ref, we2_ref, be2_ref,
                g2_ref, b2_ref, y_ref, acc):
    e = pl.program_id(1)

    @pl.when(e == 0)
    def _():
        acc[...] = jnp.zeros_like(acc)

    x1 = x1_ref[...]
    h = jnp.dot(_bf(x1), we1_ref[0],
                preferred_element_type=jnp.float32) + be1_ref[0]
    h = _gelu(h)
    eo = jnp.dot(_bf(h), we2_ref[0],
                 preferred_element_type=jnp.float32) + be2_ref[0]
    lane = jax.lax.broadcasted_iota(jnp.int32, gates_ref.shape, 1)
    gcol = jnp.sum(jnp.where(lane == e, gates_ref[...], 0.0), axis=-1, keepdims=True)
    acc[...] = acc[...] + gcol * eo

    @pl.when(e == E - 1)
    def _():
        y = acc[...] + x1
        mu = jnp.mean(y, axis=-1, keepdims=True)
        var = jnp.mean((y - mu) ** 2, axis=-1, keepdims=True)
        y_ref[...] = (y - mu) * jax.lax.rsqrt(var + 1e-5) * g2_ref[...] + b2_ref[...]


def _rel_shift(x):
    b, h, t1, t2 = x.shape
    zp = jnp.zeros((b, h, t1, 1), dtype=x.dtype)
    x = jnp.concatenate([zp, x], axis=-1)
    x = x.reshape(b, h, t2 + 1, t1)
    return x[:, :, 1:, :].reshape(b, h, t1, t2)


@functools.partial(jax.jit, static_argnums=())
def kernel(x, pos_enc, Wq, Wk, Wv, Wo, Wpos, u, v, g1, b1, g2, b2, Wg, We1, be1, We2, be2):
    f32 = jnp.float32
    x2 = x[0]
    pe = pos_enc[0]
    uf = u.reshape(1, D)
    vf = v.reshape(1, D)

    BT = 512
    qu, qv, k, val, p = pl.pallas_call(
        _proj_kernel,
        grid=(T // BT,),
        in_specs=[
            pl.BlockSpec((BT, D), lambda i: (i, 0)),
            pl.BlockSpec((BT, D), lambda i: (i, 0)),
            pl.BlockSpec((D, D), lambda i: (0, 0)),
            pl.BlockSpec((D, D), lambda i: (0, 0)),
            pl.BlockSpec((D, D), lambda i: (0, 0)),
            pl.BlockSpec((D, D), lambda i: (0, 0)),
            pl.BlockSpec((1, D), lambda i: (0, 0)),
            pl.BlockSpec((1, D), lambda i: (0, 0)),
        ],
        out_specs=[pl.BlockSpec((BT, D), lambda i: (i, 0))] * 5,
        out_shape=[jax.ShapeDtypeStruct((T, D), jnp.bfloat16)] * 5,
    )(x2, pe, Wq, Wk, Wv, Wpos, uf, vf)

    def heads(a):
        return a.reshape(T, H, DH).transpose(1, 0, 2)

    quh, qvh, kh, vh, ph = heads(qu), heads(qv), heads(k), heads(val), heads(p)

    p_roll = jnp.roll(ph, 1, axis=1)
    p_dbl = jnp.concatenate([p_roll, p_roll], axis=1)

    BQ = 256
    ao = pl.pallas_call(
        _attn_kernel,
        grid=(H, T // BQ),
        in_specs=[
            pl.BlockSpec((1, BQ, DH), lambda h, i: (h, i, 0)),
            pl.BlockSpec((1, T, DH), lambda h, i: (h, 0, 0)),
            pl.BlockSpec((1, T, DH), lambda h, i: (h, 0, 0)),
            pl.BlockSpec((1, T, DH), lambda h, i: (h, 0, 0)),
            pl.BlockSpec((1, 2 * T, DH), lambda h, i: (h, 0, 0)),
        ],
        out_specs=pl.BlockSpec((1, BQ, DH), lambda h, i: (h, i, 0)),
        out_shape=jax.ShapeDtypeStruct((H, T, DH), jnp.bfloat16),
    )(quh, qvh, kh, vh, p_dbl)

    ao_td = ao.transpose(1, 0, 2).reshape(T, D)

    x1 = pl.pallas_call(
        _postattn_kernel,
        grid=(T // BT,),
        in_specs=[
            pl.BlockSpec((BT, D), lambda i: (i, 0)),
            pl.BlockSpec((D, D), lambda i: (0, 0)),
            pl.BlockSpec((BT, D), lambda i: (i, 0)),
            pl.BlockSpec((1, D), lambda i: (0, 0)),
            pl.BlockSpec((1, D), lambda i: (0, 0)),
        ],
        out_specs=pl.BlockSpec((BT, D), lambda i: (i, 0)),
        out_shape=jax.ShapeDtypeStruct((T, D), f32),
    )(ao_td, Wo, x2, g1.reshape(1, D), b1.reshape(1, D))

    Wg_pad = jnp.zeros((D, EPAD), f32).at[:, :E].set(Wg)

    gates, loss = pl.pallas_call(
        _router_kernel,
        grid=(T // BT,),
        in_specs=[
            pl.BlockSpec((BT, D), lambda i: (i, 0)),
            pl.BlockSpec((D, EPAD), lambda i: (0, 0)),
        ],
        out_specs=[
            pl.BlockSpec((BT, EPAD), lambda i: (i, 0)),
            pl.BlockSpec((1, 1), lambda i: (0, 0)),
        ],
        out_shape=[
            jax.ShapeDtypeStruct((T, EPAD), f32),
            jax.ShapeDtypeStruct((1, 1), f32),
        ],
        scratch_shapes=[
            pltpu.VMEM((1, EPAD), f32),
            pltpu.VMEM((1, EPAD), f32),
        ],
    )(x1, Wg_pad)

    y = pl.pallas_call(
        _moe_kernel,
        grid=(T // BT, E),
        in_specs=[
            pl.BlockSpec((BT, D), lambda i, e: (i, 0)),
            pl.BlockSpec((BT, EPAD), lambda i, e: (i, 0)),
            pl.BlockSpec((1, D, FF), lambda i, e: (e, 0, 0)),
            pl.BlockSpec((1, 1, FF), lambda i, e: (e, 0, 0)),
            pl.BlockSpec((1, FF, D), lambda i, e: (e, 0, 0)),
            pl.BlockSpec((1, 1, D), lambda i, e: (e, 0, 0)),
            pl.BlockSpec((1, D), lambda i, e: (0, 0)),
            pl.BlockSpec((1, D), lambda i, e: (0, 0)),
        ],
        out_specs=pl.BlockSpec((BT, D), lambda i, e: (i, 0)),
        out_shape=jax.ShapeDtypeStruct((T, D), f32),
        scratch_shapes=[pltpu.VMEM((BT, D), f32)],
    )(x1, gates, We1.astype(jnp.bfloat16), be1.reshape(E, 1, FF),
      We2.astype(jnp.bfloat16), be2.reshape(E, 1, D),
      g2.reshape(1, D), b2.reshape(1, D))

    return (y.reshape(B, T, D), loss[0, 0])
